# Initial kernel scaffold; baseline (speedup 1.0000x reference)
import jax
import jax.numpy as jnp
from jax.experimental import pallas as pl


def kernel(x, Wr, w_up, w_down):
    raise NotImplementedError("write your pallas kernel here")


# speedup vs baseline: 1.0854x; 1.0854x over previous
import functools

import jax
import jax.numpy as jnp
from jax.experimental import pallas as pl
from jax.experimental.pallas import tpu as pltpu

E = 8
TOP_K = 2
D_MODEL = 1024
D_FF = 2048
T = 4096
CAP = int(TOP_K * T / E * 1.25)

NF = 4
BF = D_FF // NF


def _ffn_body(buf_ref, wa_ref, wb_ref, wd_ref, out_ref):
    f = pl.program_id(1)
    xb = buf_ref[...]
    a = jnp.dot(xb, wa_ref[0], preferred_element_type=jnp.float32)
    b = jnp.dot(xb, wb_ref[0], preferred_element_type=jnp.float32)
    h = a * jax.nn.sigmoid(a) * b
    contrib = jnp.dot(h, wd_ref[0], preferred_element_type=jnp.float32)

    @pl.when(f == 0)
    def _init():
        out_ref[...] = contrib

    @pl.when(f > 0)
    def _acc():
        out_ref[...] += contrib


def _ffn(buf, w_up, w_down):
    return pl.pallas_call(
        _ffn_body,
        grid=(E, NF),
        in_specs=[
            pl.BlockSpec((CAP, D_MODEL), lambda e, f: (e, 0)),
            pl.BlockSpec((1, D_MODEL, BF), lambda e, f: (e, 0, f)),
            pl.BlockSpec((1, D_MODEL, BF), lambda e, f: (e, 0, NF + f)),
            pl.BlockSpec((1, BF, D_MODEL), lambda e, f: (e, f, 0)),
        ],
        out_specs=pl.BlockSpec((CAP, D_MODEL), lambda e, f: (e, 0)),
        out_shape=jax.ShapeDtypeStruct((E * CAP, D_MODEL), jnp.float32),
        compiler_params=pltpu.CompilerParams(
            dimension_semantics=("arbitrary", "arbitrary"),
        ),
    )(buf, w_up, w_up, w_down)


def kernel(x, Wr, w_up, w_down):
    logits = x @ Wr
    probs = jax.nn.softmax(logits, axis=-1)
    topv, topi = jax.lax.top_k(probs, TOP_K)
    topv = topv / jnp.sum(topv, axis=-1, keepdims=True)
    flat_e = topi.reshape(-1)
    flat_w = topv.reshape(-1)
    onehot = jax.nn.one_hot(flat_e, E, dtype=jnp.int32)
    pos_in_e = (jnp.cumsum(onehot, axis=0) * onehot).sum(-1) - 1
    valid = pos_in_e < CAP
    token_idx = jnp.repeat(jnp.arange(T), TOP_K)
    dispatch_idx = flat_e * CAP + jnp.clip(pos_in_e, 0, CAP - 1)
    buf = jnp.zeros((E * CAP, D_MODEL), dtype=x.dtype)
    buf = buf.at[dispatch_idx].add(jnp.where(valid[:, None], x[token_idx], 0.0))
    out = _ffn(buf, w_up, w_down)
    gathered = out[dispatch_idx]
    gathered = jnp.where(valid[:, None], gathered, 0.0) * flat_w[:, None]
    y = jnp.zeros((T, D_MODEL), dtype=x.dtype).at[token_idx].add(gathered)
    return y
